# Initial kernel scaffold; baseline (speedup 1.0000x reference)
import jax
import jax.numpy as jnp
from jax.experimental import pallas as pl


def kernel(X, W_loc, b_loc):
    raise NotImplementedError("write your pallas kernel here")


# speedup vs baseline: 1.1248x; 1.1248x over previous
import functools

import jax
import jax.numpy as jnp
from jax import lax
from jax.experimental import pallas as pl
from jax.experimental.pallas import tpu as pltpu
from jax.experimental.pallas import tpu_sc as plsc

B, H, W, C = 8, 224, 224, 96
CP = 128
N = B * H * W
NW = 32
PIX_PER_W = N // NW
CHUNK = 128
NCHUNK = PIX_PER_W // CHUNK


SEG = 14
SEG_ROWS = H * W // SEG


def _prep_body(x_ref, w_ref, b_ref, th_ref, xp_ref, acc_ref):
    bb = pl.program_id(0)
    ss = pl.program_id(1)
    x = x_ref[...]
    part = jnp.sum(x, axis=0, keepdims=True)

    @pl.when(ss == 0)
    def _():
        acc_ref[pl.ds(0, 1), pl.ds(0, C)] = part

    @pl.when(ss != 0)
    def _():
        acc_ref[pl.ds(0, 1), pl.ds(0, C)] += part

    @pl.when(ss == SEG - 1)
    def _():
        pooled = acc_ref[pl.ds(0, 1), pl.ds(0, C)] / jnp.float32(H * W)
        th = jnp.dot(pooled, w_ref[...], preferred_element_type=jnp.float32)
        th_ref[pl.ds(bb, 1), :] = th + b_ref[...]

    xp_ref[:, pl.ds(0, C)] = x


def _prep_tc(flat, w16, b16):
    return pl.pallas_call(
        _prep_body,
        grid=(B, SEG),
        in_specs=[
            pl.BlockSpec((SEG_ROWS, C), lambda b, s: (b * SEG + s, 0)),
            pl.BlockSpec((C, 16), lambda b, s: (0, 0)),
            pl.BlockSpec((1, 16), lambda b, s: (0, 0)),
        ],
        out_specs=[
            pl.BlockSpec((B, 16), lambda b, s: (0, 0)),
            pl.BlockSpec((SEG_ROWS, CP), lambda b, s: (b * SEG + s, 0)),
        ],
        out_shape=[
            jax.ShapeDtypeStruct((B, 16), jnp.float32),
            jax.ShapeDtypeStruct((N, CP), jnp.float32),
        ],
        scratch_shapes=[pltpu.VMEM((8, 128), jnp.float32)],
    )(flat, w16, b16)


def _rne_bf16(v):
    u = lax.bitcast_convert_type(v, jnp.int32)
    r = u + jnp.int32(0x7FFF) + (lax.shift_right_logical(u, 16) & jnp.int32(1))
    r = r & jnp.int32(-65536)
    return lax.bitcast_convert_type(r, jnp.float32)


_mesh = plsc.VectorSubcoreMesh(core_axis_name="c", subcore_axis_name="s")


@functools.partial(
    pl.kernel,
    out_type=jax.ShapeDtypeStruct((N, C), jnp.float32),
    mesh=_mesh,
    scratch_types=[
        pltpu.VMEM((16,), jnp.float32),
        pltpu.VMEM((CHUNK,), jnp.int32),
        pltpu.VMEM((CHUNK,), jnp.int32),
        pltpu.VMEM((CHUNK,), jnp.int32),
        pltpu.VMEM((CHUNK,), jnp.int32),
        pltpu.VMEM((CHUNK,), jnp.float32),
        pltpu.VMEM((CHUNK,), jnp.float32),
        pltpu.VMEM((CHUNK,), jnp.float32),
        pltpu.VMEM((CHUNK,), jnp.float32),
        pltpu.VMEM((CHUNK, CP), jnp.float32),
        pltpu.VMEM((CHUNK, CP), jnp.float32),
        pltpu.VMEM((CHUNK, CP), jnp.float32),
        pltpu.VMEM((CHUNK, CP), jnp.float32),
        pltpu.VMEM((CHUNK, C), jnp.float32),
        pltpu.SemaphoreType.DMA,
    ],
)
def _sc_sample(flat_hbm, theta_hbm, out_hbm,
               theta_v, ia_v, ib_v, ic_v, id_v,
               wa_v, wb_v, wc_v, wd_v,
               ra_v, rb_v, rc_v, rd_v, out_v, sem):
    wid = lax.axis_index("s") * 2 + lax.axis_index("c")
    batch = lax.div(wid, jnp.int32(NW // B))
    pltpu.sync_copy(theta_hbm.at[batch], theta_v)
    tv = _rne_bf16(theta_v[...])
    t00 = tv[0]
    t01 = tv[1]
    t02 = tv[2]
    t10 = tv[3]
    t11 = tv[4]
    t12 = tv[5]
    pix0 = wid * PIX_PER_W
    ibase = batch * (H * W)
    lp0 = pix0 - ibase

    def chunk_body(g, carry):
        lp = lp0 + g * CHUNK
        for v in range(CHUNK // 16):
            pp = lp + v * 16 + lax.iota(jnp.int32, 16)
            i = lax.div(pp, jnp.int32(W))
            j = pp - i * W
            sx = j.astype(jnp.float32) * jnp.float32(1.0 / 223.0)
            x = jnp.float32(1.0) * sx - (jnp.float32(1.0) - sx)
            x = jnp.where(j == W - 1, jnp.float32(1.0), x)
            sy = i.astype(jnp.float32) * jnp.float32(1.0 / 223.0)
            y = jnp.float32(1.0) * sy - (jnp.float32(1.0) - sy)
            y = jnp.where(i == H - 1, jnp.float32(1.0), y)
            x = _rne_bf16(x)
            y = _rne_bf16(y)
            xs = t00 * x + t01 * y + t02
            ys = t10 * x + t11 * y + t12
            xf = jnp.float32(0.5) * (xs + 1.0) * jnp.float32(W)
            yf = jnp.float32(0.5) * (ys + 1.0) * jnp.float32(H)
            x0 = (xf - jnp.float32(0.5)).astype(jnp.int32)
            y0 = (yf - jnp.float32(0.5)).astype(jnp.int32)
            x1 = jnp.clip(x0 + 1, 0, W - 1)
            y1 = jnp.clip(y0 + 1, 0, H - 1)
            x0 = jnp.clip(x0, 0, W - 1)
            y0 = jnp.clip(y0, 0, H - 1)
            x0f = x0.astype(jnp.float32)
            x1f = x1.astype(jnp.float32)
            y0f = y0.astype(jnp.float32)
            y1f = y1.astype(jnp.float32)
            sl = pl.ds(v * 16, 16)
            ia_v[sl] = ibase + y0 * W + x0
            ib_v[sl] = ibase + y1 * W + x0
            ic_v[sl] = ibase + y0 * W + x1
            id_v[sl] = ibase + y1 * W + x1
            wa_v[sl] = (x1f - xf) * (y1f - yf)
            wb_v[sl] = (x1f - xf) * (yf - y0f)
            wc_v[sl] = (xf - x0f) * (y1f - yf)
            wd_v[sl] = (xf - x0f) * (yf - y0f)
        ca = pltpu.async_copy(flat_hbm.at[ia_v], ra_v, sem)
        cb = pltpu.async_copy(flat_hbm.at[ib_v], rb_v, sem)
        cc = pltpu.async_copy(flat_hbm.at[ic_v], rc_v, sem)
        cd = pltpu.async_copy(flat_hbm.at[id_v], rd_v, sem)
        ca.wait()
        cb.wait()
        cc.wait()
        cd.wait()

        def grp_body(gg, c2):
            p0 = gg * 16
            sg = pl.ds(p0, 16)
            wa16 = wa_v[sg]
            wb16 = wb_v[sg]
            wc16 = wc_v[sg]
            wd16 = wd_v[sg]
            for q in range(16):
                p = p0 + q
                a = wa16[q]
                b_ = wb16[q]
                c_ = wc16[q]
                d = wd16[q]
                for u in range(C // 16):
                    su = pl.ds(u * 16, 16)
                    out_v[p, su] = (a * ra_v[p, su] + b_ * rb_v[p, su]
                                    + c_ * rc_v[p, su] + d * rd_v[p, su])
            return c2

        lax.fori_loop(0, CHUNK // 16, grp_body, 0)
        pltpu.sync_copy(out_v, out_hbm.at[pl.ds(pix0 + g * CHUNK, CHUNK)])
        return carry

    lax.fori_loop(0, NCHUNK, chunk_body, 0)


def kernel(X, W_loc, b_loc):
    flat = X.reshape(N, C)
    w16 = jnp.pad(W_loc, ((0, 0), (0, 10)))
    b16 = jnp.pad(b_loc, (0, 10)).reshape(1, 16)
    theta16, flatp = _prep_tc(flat, w16, b16)
    out = _sc_sample(flatp, theta16)
    return out.reshape(B, H, W, C)


# speedup vs baseline: 1.8404x; 1.6362x over previous
import functools

import jax
import jax.numpy as jnp
from jax import lax
from jax.experimental import pallas as pl
from jax.experimental.pallas import tpu as pltpu
from jax.experimental.pallas import tpu_sc as plsc

B, H, W, C = 8, 224, 224, 96
CP = 128
N = B * H * W
NW = 32
PIX_PER_W = N // NW
CHUNK = 128
NCHUNK = PIX_PER_W // CHUNK


SEG = 14
SEG_H = H // SEG
SEG_ROWS = SEG_H * W


def _prep_body(x_ref, w_ref, b_ref, th_ref, xp_ref, acc_ref):
    bb = pl.program_id(0)
    ss = pl.program_id(1)
    x = x_ref[0]
    part = jnp.sum(jnp.sum(x, axis=0), axis=1, keepdims=True)

    @pl.when(ss == 0)
    def _():
        acc_ref[:, pl.ds(0, 1)] = part

    @pl.when(ss != 0)
    def _():
        acc_ref[:, pl.ds(0, 1)] += part

    @pl.when(ss == SEG - 1)
    def _():
        pooled = acc_ref[:, pl.ds(0, 1)] / jnp.float32(H * W)
        th = lax.dot_general(pooled, w_ref[...],
                             dimension_numbers=(((0,), (0,)), ((), ())),
                             preferred_element_type=jnp.float32)
        th_ref[pl.ds(bb, 1), :] = th + b_ref[...]

    for r in range(SEG_H):
        xp_ref[pl.ds(r * W, W), pl.ds(0, C)] = jnp.transpose(x[r], (1, 0))


def _prep_tc(Xt, w16, b16):
    return pl.pallas_call(
        _prep_body,
        grid=(B, SEG),
        in_specs=[
            pl.BlockSpec((1, SEG_H, C, W), lambda b, s: (b, s, 0, 0)),
            pl.BlockSpec((C, 16), lambda b, s: (0, 0)),
            pl.BlockSpec((1, 16), lambda b, s: (0, 0)),
        ],
        out_specs=[
            pl.BlockSpec((B, 16), lambda b, s: (0, 0)),
            pl.BlockSpec((SEG_ROWS, CP), lambda b, s: (b * SEG + s, 0)),
        ],
        out_shape=[
            jax.ShapeDtypeStruct((B, 16), jnp.float32),
            jax.ShapeDtypeStruct((N, CP), jnp.float32),
        ],
        scratch_shapes=[pltpu.VMEM((C, 8), jnp.float32)],
    )(Xt, w16, b16)


def _untrans_body(x_ref, o_ref):
    for r in range(SEG_H):
        o_ref[0, r] = jnp.transpose(x_ref[pl.ds(r * W, W), :], (1, 0))


def _untrans_tc(out_rows):
    return pl.pallas_call(
        _untrans_body,
        grid=(B, SEG),
        in_specs=[pl.BlockSpec((SEG_ROWS, C), lambda b, s: (b * SEG + s, 0))],
        out_specs=pl.BlockSpec((1, SEG_H, C, W), lambda b, s: (b, s, 0, 0)),
        out_shape=jax.ShapeDtypeStruct((B, H, C, W), jnp.float32),
    )(out_rows)


def _rne_bf16(v):
    u = lax.bitcast_convert_type(v, jnp.int32)
    r = u + jnp.int32(0x7FFF) + (lax.shift_right_logical(u, 16) & jnp.int32(1))
    r = r & jnp.int32(-65536)
    return lax.bitcast_convert_type(r, jnp.float32)


_mesh = plsc.VectorSubcoreMesh(core_axis_name="c", subcore_axis_name="s")


@functools.partial(
    pl.kernel,
    out_type=jax.ShapeDtypeStruct((N, C), jnp.float32),
    mesh=_mesh,
    scratch_types=[
        pltpu.VMEM((16,), jnp.float32),
        pltpu.VMEM((CHUNK,), jnp.int32),
        pltpu.VMEM((CHUNK,), jnp.int32),
        pltpu.VMEM((CHUNK,), jnp.int32),
        pltpu.VMEM((CHUNK,), jnp.int32),
        pltpu.VMEM((CHUNK,), jnp.float32),
        pltpu.VMEM((CHUNK,), jnp.float32),
        pltpu.VMEM((CHUNK,), jnp.float32),
        pltpu.VMEM((CHUNK,), jnp.float32),
        pltpu.VMEM((CHUNK, CP), jnp.float32),
        pltpu.VMEM((CHUNK, CP), jnp.float32),
        pltpu.VMEM((CHUNK, CP), jnp.float32),
        pltpu.VMEM((CHUNK, CP), jnp.float32),
        pltpu.VMEM((CHUNK, C), jnp.float32),
        pltpu.SemaphoreType.DMA,
    ],
)
def _sc_sample(flat_hbm, theta_hbm, out_hbm,
               theta_v, ia_v, ib_v, ic_v, id_v,
               wa_v, wb_v, wc_v, wd_v,
               ra_v, rb_v, rc_v, rd_v, out_v, sem):
    wid = lax.axis_index("s") * 2 + lax.axis_index("c")
    batch = lax.div(wid, jnp.int32(NW // B))
    pltpu.sync_copy(theta_hbm.at[batch], theta_v)
    tv = _rne_bf16(theta_v[...])
    t00 = tv[0]
    t01 = tv[1]
    t02 = tv[2]
    t10 = tv[3]
    t11 = tv[4]
    t12 = tv[5]
    pix0 = wid * PIX_PER_W
    ibase = batch * (H * W)
    lp0 = pix0 - ibase

    def chunk_body(g, carry):
        lp = lp0 + g * CHUNK
        for v in range(CHUNK // 16):
            pp = lp + v * 16 + lax.iota(jnp.int32, 16)
            i = lax.div(pp, jnp.int32(W))
            j = pp - i * W
            sx = j.astype(jnp.float32) * jnp.float32(1.0 / 223.0)
            x = jnp.float32(1.0) * sx - (jnp.float32(1.0) - sx)
            x = jnp.where(j == W - 1, jnp.float32(1.0), x)
            sy = i.astype(jnp.float32) * jnp.float32(1.0 / 223.0)
            y = jnp.float32(1.0) * sy - (jnp.float32(1.0) - sy)
            y = jnp.where(i == H - 1, jnp.float32(1.0), y)
            x = _rne_bf16(x)
            y = _rne_bf16(y)
            xs = t00 * x + t01 * y + t02
            ys = t10 * x + t11 * y + t12
            xf = jnp.float32(0.5) * (xs + 1.0) * jnp.float32(W)
            yf = jnp.float32(0.5) * (ys + 1.0) * jnp.float32(H)
            x0 = (xf - jnp.float32(0.5)).astype(jnp.int32)
            y0 = (yf - jnp.float32(0.5)).astype(jnp.int32)
            x1 = jnp.clip(x0 + 1, 0, W - 1)
            y1 = jnp.clip(y0 + 1, 0, H - 1)
            x0 = jnp.clip(x0, 0, W - 1)
            y0 = jnp.clip(y0, 0, H - 1)
            x0f = x0.astype(jnp.float32)
            x1f = x1.astype(jnp.float32)
            y0f = y0.astype(jnp.float32)
            y1f = y1.astype(jnp.float32)
            sl = pl.ds(v * 16, 16)
            ia_v[sl] = ibase + y0 * W + x0
            ib_v[sl] = ibase + y1 * W + x0
            ic_v[sl] = ibase + y0 * W + x1
            id_v[sl] = ibase + y1 * W + x1
            wa_v[sl] = (x1f - xf) * (y1f - yf)
            wb_v[sl] = (x1f - xf) * (yf - y0f)
            wc_v[sl] = (xf - x0f) * (y1f - yf)
            wd_v[sl] = (xf - x0f) * (yf - y0f)
        ca = pltpu.async_copy(flat_hbm.at[ia_v], ra_v, sem)
        cb = pltpu.async_copy(flat_hbm.at[ib_v], rb_v, sem)
        cc = pltpu.async_copy(flat_hbm.at[ic_v], rc_v, sem)
        cd = pltpu.async_copy(flat_hbm.at[id_v], rd_v, sem)
        ca.wait()
        cb.wait()
        cc.wait()
        cd.wait()

        def grp_body(gg, c2):
            p0 = gg * 16
            sg = pl.ds(p0, 16)
            wa16 = wa_v[sg]
            wb16 = wb_v[sg]
            wc16 = wc_v[sg]
            wd16 = wd_v[sg]
            for q in range(16):
                p = p0 + q
                a = wa16[q]
                b_ = wb16[q]
                c_ = wc16[q]
                d = wd16[q]
                for u in range(C // 16):
                    su = pl.ds(u * 16, 16)
                    out_v[p, su] = (a * ra_v[p, su] + b_ * rb_v[p, su]
                                    + c_ * rc_v[p, su] + d * rd_v[p, su])
            return c2

        lax.fori_loop(0, CHUNK // 16, grp_body, 0)
        pltpu.sync_copy(out_v, out_hbm.at[pl.ds(pix0 + g * CHUNK, CHUNK)])
        return carry

    lax.fori_loop(0, NCHUNK, chunk_body, 0)


def kernel(X, W_loc, b_loc):
    Xt = jnp.transpose(X, (0, 1, 3, 2))
    w16 = jnp.pad(W_loc, ((0, 0), (0, 10)))
    b16 = jnp.pad(b_loc, (0, 10)).reshape(1, 16)
    theta16, flatp = _prep_tc(Xt, w16, b16)
    out = _sc_sample(flatp, theta16)
    out_t = _untrans_tc(out)
    return jnp.transpose(out_t, (0, 1, 3, 2))


# speedup vs baseline: 2.2613x; 1.2287x over previous
import functools

import jax
import jax.numpy as jnp
from jax import lax
from jax.experimental import pallas as pl
from jax.experimental.pallas import tpu as pltpu
from jax.experimental.pallas import tpu_sc as plsc

B, H, W, C = 8, 224, 224, 96
CP = 128
N = B * H * W
NW = 32
PIX_PER_W = N // NW
CHUNK = 64
NCHUNK = PIX_PER_W // CHUNK


SEG = 14
SEG_H = H // SEG
SEG_ROWS = SEG_H * W


def _prep_body(x_ref, w_ref, b_ref, th_ref, xp_ref, acc_ref):
    bb = pl.program_id(0)
    ss = pl.program_id(1)
    x = x_ref[0]
    part = jnp.sum(jnp.sum(x, axis=0), axis=1, keepdims=True)

    @pl.when(ss == 0)
    def _():
        acc_ref[:, pl.ds(0, 1)] = part

    @pl.when(ss != 0)
    def _():
        acc_ref[:, pl.ds(0, 1)] += part

    @pl.when(ss == SEG - 1)
    def _():
        pooled = acc_ref[:, pl.ds(0, 1)] / jnp.float32(H * W)
        th = lax.dot_general(pooled, w_ref[...],
                             dimension_numbers=(((0,), (0,)), ((), ())),
                             preferred_element_type=jnp.float32)
        th_ref[pl.ds(bb, 1), :] = th + b_ref[...]

    for r in range(SEG_H):
        xp_ref[pl.ds(r * W, W), pl.ds(0, C)] = jnp.transpose(x[r], (1, 0))


def _prep_tc(Xt, w16, b16):
    return pl.pallas_call(
        _prep_body,
        grid=(B, SEG),
        in_specs=[
            pl.BlockSpec((1, SEG_H, C, W), lambda b, s: (b, s, 0, 0)),
            pl.BlockSpec((C, 16), lambda b, s: (0, 0)),
            pl.BlockSpec((1, 16), lambda b, s: (0, 0)),
        ],
        out_specs=[
            pl.BlockSpec((B, 16), lambda b, s: (0, 0)),
            pl.BlockSpec((SEG_ROWS, CP), lambda b, s: (b * SEG + s, 0)),
        ],
        out_shape=[
            jax.ShapeDtypeStruct((B, 16), jnp.float32),
            jax.ShapeDtypeStruct((N, CP), jnp.float32),
        ],
        scratch_shapes=[pltpu.VMEM((C, 8), jnp.float32)],
    )(Xt, w16, b16)


def _untrans_body(x_ref, o_ref):
    for r in range(SEG_H):
        o_ref[0, r] = jnp.transpose(x_ref[pl.ds(r * W, W), :], (1, 0))


def _untrans_tc(out_rows):
    return pl.pallas_call(
        _untrans_body,
        grid=(B, SEG),
        in_specs=[pl.BlockSpec((SEG_ROWS, C), lambda b, s: (b * SEG + s, 0))],
        out_specs=pl.BlockSpec((1, SEG_H, C, W), lambda b, s: (b, s, 0, 0)),
        out_shape=jax.ShapeDtypeStruct((B, H, C, W), jnp.float32),
    )(out_rows)


def _rne_bf16(v):
    u = lax.bitcast_convert_type(v, jnp.int32)
    r = u + jnp.int32(0x7FFF) + (lax.shift_right_logical(u, 16) & jnp.int32(1))
    r = r & jnp.int32(-65536)
    return lax.bitcast_convert_type(r, jnp.float32)


_mesh = plsc.VectorSubcoreMesh(core_axis_name="c", subcore_axis_name="s")


NBUF = 2


@functools.partial(
    pl.kernel,
    out_type=jax.ShapeDtypeStruct((N, C), jnp.float32),
    mesh=_mesh,
    scratch_types=(
        [pltpu.VMEM((16,), jnp.float32)]
        + [pltpu.VMEM((CHUNK,), jnp.int32) for _ in range(4 * NBUF)]
        + [pltpu.VMEM((CHUNK,), jnp.float32) for _ in range(4 * NBUF)]
        + [pltpu.VMEM((CHUNK, CP), jnp.float32) for _ in range(4 * NBUF)]
        + [pltpu.VMEM((CHUNK, C), jnp.float32)]
        + [pltpu.SemaphoreType.DMA for _ in range(NBUF)]
    ),
)
def _sc_sample(flat_hbm, theta_hbm, out_hbm, theta_v, *sc):
    idx_bufs = [sc[0 + 4 * p: 4 + 4 * p] for p in range(NBUF)]
    w_bufs = [sc[8 + 4 * p: 12 + 4 * p] for p in range(NBUF)]
    row_bufs = [sc[16 + 4 * p: 20 + 4 * p] for p in range(NBUF)]
    out_v = sc[24]
    sems = sc[25:25 + NBUF]

    wid = lax.axis_index("s") * 2 + lax.axis_index("c")
    batch = lax.div(wid, jnp.int32(NW // B))
    pltpu.sync_copy(theta_hbm.at[batch], theta_v)
    tv = _rne_bf16(theta_v[...])
    t00 = tv[0]
    t01 = tv[1]
    t02 = tv[2]
    t10 = tv[3]
    t11 = tv[4]
    t12 = tv[5]
    pix0 = wid * PIX_PER_W
    ibase = batch * (H * W)
    lp0 = pix0 - ibase

    def fire(g, p):
        ia_v, ib_v, ic_v, id_v = idx_bufs[p]
        wa_v, wb_v, wc_v, wd_v = w_bufs[p]
        lp = lp0 + g * CHUNK
        for v in range(CHUNK // 16):
            pp = lp + v * 16 + lax.iota(jnp.int32, 16)
            i = lax.div(pp, jnp.int32(W))
            j = pp - i * W
            sx = j.astype(jnp.float32) * jnp.float32(1.0 / 223.0)
            x = jnp.float32(1.0) * sx - (jnp.float32(1.0) - sx)
            x = jnp.where(j == W - 1, jnp.float32(1.0), x)
            sy = i.astype(jnp.float32) * jnp.float32(1.0 / 223.0)
            y = jnp.float32(1.0) * sy - (jnp.float32(1.0) - sy)
            y = jnp.where(i == H - 1, jnp.float32(1.0), y)
            x = _rne_bf16(x)
            y = _rne_bf16(y)
            xs = t00 * x + t01 * y + t02
            ys = t10 * x + t11 * y + t12
            xf = jnp.float32(0.5) * (xs + 1.0) * jnp.float32(W)
            yf = jnp.float32(0.5) * (ys + 1.0) * jnp.float32(H)
            x0 = (xf - jnp.float32(0.5)).astype(jnp.int32)
            y0 = (yf - jnp.float32(0.5)).astype(jnp.int32)
            x1 = jnp.clip(x0 + 1, 0, W - 1)
            y1 = jnp.clip(y0 + 1, 0, H - 1)
            x0 = jnp.clip(x0, 0, W - 1)
            y0 = jnp.clip(y0, 0, H - 1)
            x0f = x0.astype(jnp.float32)
            x1f = x1.astype(jnp.float32)
            y0f = y0.astype(jnp.float32)
            y1f = y1.astype(jnp.float32)
            sl = pl.ds(v * 16, 16)
            ia_v[sl] = ibase + y0 * W + x0
            ib_v[sl] = ibase + y1 * W + x0
            ic_v[sl] = ibase + y0 * W + x1
            id_v[sl] = ibase + y1 * W + x1
            wa_v[sl] = (x1f - xf) * (y1f - yf)
            wb_v[sl] = (x1f - xf) * (yf - y0f)
            wc_v[sl] = (xf - x0f) * (y1f - yf)
            wd_v[sl] = (xf - x0f) * (yf - y0f)
        for k in range(4):
            pltpu.async_copy(flat_hbm.at[idx_bufs[p][k]], row_bufs[p][k],
                             sems[p])

    def drain(p):
        for k in range(4):
            pltpu.make_async_copy(flat_hbm.at[pl.ds(0, CHUNK)],
                                  row_bufs[p][k], sems[p]).wait()

    def blend_store(g, p):
        ra_v, rb_v, rc_v, rd_v = row_bufs[p]
        wa_v, wb_v, wc_v, wd_v = w_bufs[p]

        def grp_body(gg, c2):
            p0 = gg * 16
            sg = pl.ds(p0, 16)
            wa16 = wa_v[sg]
            wb16 = wb_v[sg]
            wc16 = wc_v[sg]
            wd16 = wd_v[sg]
            for q in range(16):
                pq = p0 + q
                a = wa16[q]
                b_ = wb16[q]
                c_ = wc16[q]
                d = wd16[q]
                for u in range(C // 16):
                    su = pl.ds(u * 16, 16)
                    out_v[pq, su] = (a * ra_v[pq, su] + b_ * rb_v[pq, su]
                                     + c_ * rc_v[pq, su] + d * rd_v[pq, su])
            return c2

        lax.fori_loop(0, CHUNK // 16, grp_body, 0)
        pltpu.sync_copy(out_v, out_hbm.at[pl.ds(pix0 + g * CHUNK, CHUNK)])

    NPAIR = NCHUNK // 2
    fire(0, 0)

    def pair_body(h, carry):
        g0 = 2 * h
        fire(g0 + 1, 1)
        drain(0)
        blend_store(g0, 0)

        @pl.when(h < NPAIR - 1)
        def _():
            fire(g0 + 2, 0)

        drain(1)
        blend_store(g0 + 1, 1)
        return carry

    lax.fori_loop(0, NPAIR, pair_body, 0)


def kernel(X, W_loc, b_loc):
    Xt = jnp.transpose(X, (0, 1, 3, 2))
    w16 = jnp.pad(W_loc, ((0, 0), (0, 10)))
    b16 = jnp.pad(b_loc, (0, 10)).reshape(1, 16)
    theta16, flatp = _prep_tc(Xt, w16, b16)
    out = _sc_sample(flatp, theta16)
    out_t = _untrans_tc(out)
    return jnp.transpose(out_t, (0, 1, 3, 2))


# speedup vs baseline: 2.3088x; 1.0210x over previous
import functools

import jax
import jax.numpy as jnp
from jax import lax
from jax.experimental import pallas as pl
from jax.experimental.pallas import tpu as pltpu
from jax.experimental.pallas import tpu_sc as plsc

B, H, W, C = 8, 224, 224, 96
CP = 128
N = B * H * W
NW = 32
PIX_PER_W = N // NW
CHUNK = 64
NCHUNK = PIX_PER_W // CHUNK


SEG = 14
SEG_H = H // SEG
SEG_ROWS = SEG_H * W


def _prep_body(x_ref, w_ref, b_ref, th_ref, xp_ref, acc_ref):
    bb = pl.program_id(0)
    ss = pl.program_id(1)
    x = x_ref[0]
    part = jnp.sum(jnp.sum(x, axis=0), axis=1, keepdims=True)

    @pl.when(ss == 0)
    def _():
        acc_ref[:, pl.ds(0, 1)] = part

    @pl.when(ss != 0)
    def _():
        acc_ref[:, pl.ds(0, 1)] += part

    @pl.when(ss == SEG - 1)
    def _():
        pooled = acc_ref[:, pl.ds(0, 1)] / jnp.float32(H * W)
        th = lax.dot_general(pooled, w_ref[...],
                             dimension_numbers=(((0,), (0,)), ((), ())),
                             preferred_element_type=jnp.float32)
        th_ref[pl.ds(bb, 1), :] = th + b_ref[...]

    for r in range(SEG_H):
        xp_ref[pl.ds(r * W, W), pl.ds(0, C)] = jnp.transpose(x[r], (1, 0))


def _prep_tc(Xt, w16, b16):
    return pl.pallas_call(
        _prep_body,
        grid=(B, SEG),
        in_specs=[
            pl.BlockSpec((1, SEG_H, C, W), lambda b, s: (b, s, 0, 0)),
            pl.BlockSpec((C, 16), lambda b, s: (0, 0)),
            pl.BlockSpec((1, 16), lambda b, s: (0, 0)),
        ],
        out_specs=[
            pl.BlockSpec((B, 16), lambda b, s: (0, 0)),
            pl.BlockSpec((SEG_ROWS, CP), lambda b, s: (b * SEG + s, 0)),
        ],
        out_shape=[
            jax.ShapeDtypeStruct((B, 16), jnp.float32),
            jax.ShapeDtypeStruct((N, CP), jnp.float32),
        ],
        scratch_shapes=[pltpu.VMEM((C, 8), jnp.float32)],
    )(Xt, w16, b16)


def _untrans_body(x_ref, o_ref):
    for r in range(SEG_H):
        o_ref[0, r] = jnp.transpose(x_ref[pl.ds(r * W, W), :], (1, 0))


def _untrans_tc(out_rows):
    return pl.pallas_call(
        _untrans_body,
        grid=(B, SEG),
        in_specs=[pl.BlockSpec((SEG_ROWS, C), lambda b, s: (b * SEG + s, 0))],
        out_specs=pl.BlockSpec((1, SEG_H, C, W), lambda b, s: (b, s, 0, 0)),
        out_shape=jax.ShapeDtypeStruct((B, H, C, W), jnp.float32),
    )(out_rows)


def _rne_bf16(v):
    u = lax.bitcast_convert_type(v, jnp.int32)
    r = u + jnp.int32(0x7FFF) + (lax.shift_right_logical(u, 16) & jnp.int32(1))
    r = r & jnp.int32(-65536)
    return lax.bitcast_convert_type(r, jnp.float32)


_mesh = plsc.VectorSubcoreMesh(core_axis_name="c", subcore_axis_name="s")


NBUF = 2


@functools.partial(
    pl.kernel,
    out_type=jax.ShapeDtypeStruct((N, C), jnp.float32),
    mesh=_mesh,
    scratch_types=(
        [pltpu.VMEM((16,), jnp.float32)]
        + [pltpu.VMEM((CHUNK,), jnp.int32) for _ in range(4 * NBUF)]
        + [pltpu.VMEM((CHUNK,), jnp.float32) for _ in range(4 * NBUF)]
        + [pltpu.VMEM((CHUNK, CP), jnp.float32) for _ in range(4 * NBUF)]
        + [pltpu.VMEM((CHUNK, C), jnp.float32) for _ in range(NBUF)]
        + [pltpu.SemaphoreType.DMA for _ in range(2 * NBUF)]
    ),
)
def _sc_sample(flat_hbm, theta_hbm, out_hbm, theta_v, *sc):
    idx_bufs = [sc[0 + 4 * p: 4 + 4 * p] for p in range(NBUF)]
    w_bufs = [sc[8 + 4 * p: 12 + 4 * p] for p in range(NBUF)]
    row_bufs = [sc[16 + 4 * p: 20 + 4 * p] for p in range(NBUF)]
    out_bufs = sc[24:24 + NBUF]
    sems = sc[24 + NBUF:24 + 2 * NBUF]
    osems = sc[24 + 2 * NBUF:24 + 3 * NBUF]

    wid = lax.axis_index("s") * 2 + lax.axis_index("c")
    batch = lax.div(wid, jnp.int32(NW // B))
    pltpu.sync_copy(theta_hbm.at[batch], theta_v)
    tv = _rne_bf16(theta_v[...])
    t00 = tv[0]
    t01 = tv[1]
    t02 = tv[2]
    t10 = tv[3]
    t11 = tv[4]
    t12 = tv[5]
    pix0 = wid * PIX_PER_W
    ibase = batch * (H * W)
    lp0 = pix0 - ibase

    def fire(g, p):
        ia_v, ib_v, ic_v, id_v = idx_bufs[p]
        wa_v, wb_v, wc_v, wd_v = w_bufs[p]
        lp = lp0 + g * CHUNK
        for v in range(CHUNK // 16):
            pp = lp + v * 16 + lax.iota(jnp.int32, 16)
            i = lax.div(pp, jnp.int32(W))
            j = pp - i * W
            sx = j.astype(jnp.float32) * jnp.float32(1.0 / 223.0)
            x = jnp.float32(1.0) * sx - (jnp.float32(1.0) - sx)
            x = jnp.where(j == W - 1, jnp.float32(1.0), x)
            sy = i.astype(jnp.float32) * jnp.float32(1.0 / 223.0)
            y = jnp.float32(1.0) * sy - (jnp.float32(1.0) - sy)
            y = jnp.where(i == H - 1, jnp.float32(1.0), y)
            x = _rne_bf16(x)
            y = _rne_bf16(y)
            xs = t00 * x + t01 * y + t02
            ys = t10 * x + t11 * y + t12
            xf = jnp.float32(0.5) * (xs + 1.0) * jnp.float32(W)
            yf = jnp.float32(0.5) * (ys + 1.0) * jnp.float32(H)
            x0 = (xf - jnp.float32(0.5)).astype(jnp.int32)
            y0 = (yf - jnp.float32(0.5)).astype(jnp.int32)
            x1 = jnp.clip(x0 + 1, 0, W - 1)
            y1 = jnp.clip(y0 + 1, 0, H - 1)
            x0 = jnp.clip(x0, 0, W - 1)
            y0 = jnp.clip(y0, 0, H - 1)
            x0f = x0.astype(jnp.float32)
            x1f = x1.astype(jnp.float32)
            y0f = y0.astype(jnp.float32)
            y1f = y1.astype(jnp.float32)
            sl = pl.ds(v * 16, 16)
            ia_v[sl] = ibase + y0 * W + x0
            ib_v[sl] = ibase + y1 * W + x0
            ic_v[sl] = ibase + y0 * W + x1
            id_v[sl] = ibase + y1 * W + x1
            wa_v[sl] = (x1f - xf) * (y1f - yf)
            wb_v[sl] = (x1f - xf) * (yf - y0f)
            wc_v[sl] = (xf - x0f) * (y1f - yf)
            wd_v[sl] = (xf - x0f) * (yf - y0f)
        for k in range(4):
            pltpu.async_copy(flat_hbm.at[idx_bufs[p][k]], row_bufs[p][k],
                             sems[p])

    def drain(p):
        for k in range(4):
            pltpu.make_async_copy(flat_hbm.at[pl.ds(0, CHUNK)],
                                  row_bufs[p][k], sems[p]).wait()

    def blend_store(g, p):
        ra_v, rb_v, rc_v, rd_v = row_bufs[p]
        wa_v, wb_v, wc_v, wd_v = w_bufs[p]
        out_v = out_bufs[p]

        @pl.when(g >= NBUF)
        def _():
            pltpu.make_async_copy(out_hbm.at[pl.ds(0, CHUNK)],
                                  out_v, osems[p]).wait()

        def grp_body(gg, c2):
            p0 = gg * 16
            sg = pl.ds(p0, 16)
            wa16 = wa_v[sg]
            wb16 = wb_v[sg]
            wc16 = wc_v[sg]
            wd16 = wd_v[sg]
            for q in range(16):
                pq = p0 + q
                a = wa16[q]
                b_ = wb16[q]
                c_ = wc16[q]
                d = wd16[q]
                for u in range(C // 16):
                    su = pl.ds(u * 16, 16)
                    out_v[pq, su] = (a * ra_v[pq, su] + b_ * rb_v[pq, su]
                                     + c_ * rc_v[pq, su] + d * rd_v[pq, su])
            return c2

        lax.fori_loop(0, CHUNK // 16, grp_body, 0)
        pltpu.async_copy(out_v, out_hbm.at[pl.ds(pix0 + g * CHUNK, CHUNK)],
                         osems[p])

    NPAIR = NCHUNK // 2
    fire(0, 0)

    def pair_body(h, carry):
        g0 = 2 * h
        fire(g0 + 1, 1)
        drain(0)
        blend_store(g0, 0)

        @pl.when(h < NPAIR - 1)
        def _():
            fire(g0 + 2, 0)

        drain(1)
        blend_store(g0 + 1, 1)
        return carry

    lax.fori_loop(0, NPAIR, pair_body, 0)
    for p in range(NBUF):
        pltpu.make_async_copy(out_hbm.at[pl.ds(0, CHUNK)],
                              out_bufs[p], osems[p]).wait()


def kernel(X, W_loc, b_loc):
    Xt = jnp.transpose(X, (0, 1, 3, 2))
    w16 = jnp.pad(W_loc, ((0, 0), (0, 10)))
    b16 = jnp.pad(b_loc, (0, 10)).reshape(1, 16)
    theta16, flatp = _prep_tc(Xt, w16, b16)
    out = _sc_sample(flatp, theta16)
    out_t = _untrans_tc(out)
    return jnp.transpose(out_t, (0, 1, 3, 2))
